# Initial kernel scaffold; baseline (speedup 1.0000x reference)
import jax
import jax.numpy as jnp
from jax.experimental import pallas as pl


def kernel(X, rows, cols, vals):
    raise NotImplementedError("write your pallas kernel here")


# speedup vs baseline: 3.0666x; 3.0666x over previous
import functools

import jax
import jax.numpy as jnp
from jax import lax
from jax.experimental import pallas as pl
from jax.experimental.pallas import tpu as pltpu
from jax.experimental.pallas import tpu_sc as plsc

NC = 2
NS = 16
L = 16
NT = NC * NS
K = 128
R = 1024


def _transpose_tc(x):
    b, f = x.shape
    blk = 2048

    def body(x_ref, o_ref):
        o_ref[...] = x_ref[...].T

    return pl.pallas_call(
        body,
        grid=(f // blk,),
        in_specs=[pl.BlockSpec((b, blk), lambda i: (0, i))],
        out_specs=pl.BlockSpec((blk, b), lambda i: (i, 0)),
        out_shape=jax.ShapeDtypeStruct((f, b), jnp.float32),
    )(x)


def _combine_tc(partials):
    nc, r, b = partials.shape

    def body(p_ref, o_ref):
        acc = p_ref[0]
        for i in range(1, nc):
            acc = acc + p_ref[i]
        o_ref[...] = acc.T

    return pl.pallas_call(
        body,
        out_shape=jax.ShapeDtypeStruct((b, r), jnp.float32),
    )(partials)


def _sc_spmm(xt, rows3, cols3, vals3, n_chunks, batch):
    mesh = plsc.VectorSubcoreMesh(
        core_axis_name="c", subcore_axis_name="s",
        num_cores=NC, num_subcores=NS,
    )
    rows_per_tile = R // NS

    @functools.partial(
        pl.kernel,
        out_type=jax.ShapeDtypeStruct((NC, R, batch), jnp.float32),
        mesh=mesh,
        compiler_params=pltpu.CompilerParams(use_tc_tiling_on_sc=False),
        scratch_types=[
            pltpu.VMEM((n_chunks, K), jnp.int32),
            pltpu.VMEM((n_chunks, K), jnp.int32),
            pltpu.VMEM((K, batch), jnp.float32),
            pltpu.VMEM((K, batch), jnp.float32),
            pltpu.VMEM((n_chunks, K), jnp.float32),
            pltpu.VMEM_SHARED((R, batch), jnp.float32),
            pltpu.SemaphoreType.DMA,
            pltpu.SemaphoreType.DMA,
        ],
    )
    def k(xt_hbm, rows_hbm, cols_hbm, vals_hbm, out_hbm,
          cols_v, rows_v, buf_a, buf_b, vals_v, acc, sem_a, sem_b):
        c = lax.axis_index("c")
        s = lax.axis_index("s")
        w = c * NS + s

        pltpu.sync_copy(cols_hbm.at[w], cols_v)
        pltpu.sync_copy(rows_hbm.at[w], rows_v)
        pltpu.sync_copy(vals_hbm.at[w], vals_v)

        @pl.loop(0, rows_per_tile)
        def _(i):
            for kk in range(batch // L):
                buf_a[i, pl.ds(kk * L, L)] = jnp.zeros((L,), jnp.float32)

        pltpu.sync_copy(
            buf_a.at[pl.ds(0, rows_per_tile)],
            acc.at[pl.ds(s * rows_per_tile, rows_per_tile)],
        )
        plsc.subcore_barrier()

        def gather_start(j, buf, sem):
            pltpu.async_copy(xt_hbm.at[cols_v.at[j]], buf, sem)

        def gather_wait(j, buf, sem):
            pltpu.make_async_copy(xt_hbm.at[cols_v.at[j]], buf, sem).wait()

        def scale(buf, j):
            @pl.loop(0, K // L)
            def _(g):
                vv = vals_v[j, pl.ds(g * L, L)]
                for t in range(L):
                    v = vv[t]
                    i = g * L + t
                    for kk in range(batch // L):
                        sl = pl.ds(kk * L, L)
                        buf[i, sl] = buf[i, sl] * v

        def scatter_add(buf, j):
            pltpu.sync_copy(buf, acc.at[rows_v.at[j]], add=True)

        gather_start(0, buf_a, sem_a)
        gather_start(1, buf_b, sem_b)

        @pl.loop(0, n_chunks - 1, step=2)
        def _(j):
            gather_wait(j, buf_a, sem_a)
            scale(buf_a, j)
            scatter_add(buf_a, j)
            gather_start(j + 2, buf_a, sem_a)

            gather_wait(j + 1, buf_b, sem_b)
            scale(buf_b, j + 1)
            scatter_add(buf_b, j + 1)

            @pl.when(j + 3 < n_chunks)
            def _():
                gather_start(j + 3, buf_b, sem_b)

        last = n_chunks - 1
        gather_wait(last, buf_a, sem_a)
        scale(buf_a, last)
        scatter_add(buf_a, last)

        plsc.subcore_barrier()
        pltpu.sync_copy(
            acc.at[pl.ds(s * rows_per_tile, rows_per_tile)],
            out_hbm.at[c, pl.ds(s * rows_per_tile, rows_per_tile)],
        )

    return k(xt, rows3, cols3, vals3)


def kernel(X, rows, cols, vals):
    if X.ndim > 2:
        X = X.reshape(X.shape[0], -1)
    batch = X.shape[0]
    n = rows.shape[0]

    n_chunks = -(-n // (NT * K))
    if n_chunks % 2 == 0:
        n_chunks += 1
    pad = NT * K * n_chunks - n
    rows_p = jnp.concatenate([rows.astype(jnp.int32), jnp.zeros((pad,), jnp.int32)])
    cols_p = jnp.concatenate([cols.astype(jnp.int32), jnp.zeros((pad,), jnp.int32)])
    vals_p = jnp.concatenate([vals, jnp.zeros((pad,), jnp.float32)])
    rows3 = rows_p.reshape(NT, n_chunks, K)
    cols3 = cols_p.reshape(NT, n_chunks, K)
    vals3 = vals_p.reshape(NT, n_chunks, K)

    xt = _transpose_tc(X)
    partials = _sc_spmm(xt, rows3, cols3, vals3, n_chunks, batch)
    return _combine_tc(partials)
